# baseline (speedup 1.0000x reference)
import functools

import jax
import jax.numpy as jnp
from jax import lax
from jax.experimental import pallas as pl
from jax.experimental.pallas import tpu as pltpu
from jax.experimental.pallas import tpu_sc as plsc

N = 10000
E = 320000
D_IN = 128
H0 = 32
H1 = 8
ZETA = 1.01

NC = 2
NS = 16
NW = NC * NS
EPT = E // NW
WIN_B = 1000
NWIN_B = EPT // WIN_B
WIN = 2000
NWIN = EPT // WIN

_MESH = plsc.VectorSubcoreMesh(
    core_axis_name="c", subcore_axis_name="s", num_cores=NC, num_subcores=NS
)
_SC_PARAMS = pltpu.CompilerParams(
    use_tc_tiling_on_sc=False, needs_layout_passes=False
)


def _wid(cid, sid):
    return cid * NS + sid


@functools.partial(
    pl.kernel,
    out_type=jax.ShapeDtypeStruct((NC, N), jnp.float32),
    mesh=_MESH,
    compiler_params=_SC_PARAMS,
    scratch_types=[
        pltpu.VMEM((EPT,), jnp.int32),
        pltpu.VMEM((N,), jnp.float32),
        pltpu.VMEM_SHARED((N,), jnp.float32),
    ],
)
def _sc_deg(col_hbm, ones_hbm, zeros_hbm, deg_out, colw, onesv, acc):
    cid = lax.axis_index("c")
    sid = lax.axis_index("s")
    base = _wid(cid, sid) * EPT
    pltpu.sync_copy(col_hbm.at[pl.ds(base, EPT)], colw)
    pltpu.sync_copy(ones_hbm, onesv)

    @pl.when(sid == 0)
    def _():
        pltpu.sync_copy(zeros_hbm, acc)

    plsc.subcore_barrier()
    pltpu.sync_copy(onesv, acc.at[colw], add=True)
    plsc.subcore_barrier()

    @pl.when(sid == 0)
    def _():
        pltpu.sync_copy(acc, deg_out.at[cid])


@functools.partial(
    pl.kernel,
    out_type=jax.ShapeDtypeStruct((NC, N, H0), jnp.float32),
    mesh=_MESH,
    compiler_params=_SC_PARAMS,
    scratch_types=[
        pltpu.VMEM((WIN_B,), jnp.int32),
        pltpu.VMEM((WIN_B,), jnp.int32),
        pltpu.VMEM((WIN_B, H0), jnp.float32),
        pltpu.VMEM((WIN_B,), jnp.int32),
        pltpu.VMEM((WIN_B,), jnp.int32),
        pltpu.VMEM((WIN_B, H0), jnp.float32),
        pltpu.VMEM_SHARED((N, H0), jnp.float32),
        pltpu.SemaphoreType.DMA,
        pltpu.SemaphoreType.DMA,
    ],
)
def _sc_agg32(row_hbm, col_hbm, tmp_hbm, zeros_hbm, s_out, roww0, colw0,
              rows0, roww1, colw1, rows1, acc, sem0, sem1):
    cid = lax.axis_index("c")
    sid = lax.axis_index("s")
    base = _wid(cid, sid) * EPT
    rw = [roww0, roww1]
    cw = [colw0, colw1]
    rb = [rows0, rows1]
    sems = [sem0, sem1]

    @pl.when(sid == 0)
    def _():
        pltpu.sync_copy(zeros_hbm, acc)

    plsc.subcore_barrier()
    pltpu.sync_copy(row_hbm.at[pl.ds(base, WIN_B)], roww0)
    copies = {0: pltpu.async_copy(tmp_hbm.at[roww0], rows0, sem0)}
    pltpu.sync_copy(col_hbm.at[pl.ds(base, WIN_B)], colw0)
    for w in range(NWIN_B):
        cur, nxt = w % 2, (w + 1) % 2
        if w + 1 < NWIN_B:
            b1 = base + (w + 1) * WIN_B
            pltpu.sync_copy(row_hbm.at[pl.ds(b1, WIN_B)], rw[nxt])
            copies[w + 1] = pltpu.async_copy(tmp_hbm.at[rw[nxt]], rb[nxt],
                                             sems[nxt])
            pltpu.sync_copy(col_hbm.at[pl.ds(b1, WIN_B)], cw[nxt])
        copies[w].wait()
        pltpu.sync_copy(rb[cur], acc.at[cw[cur]], add=True)
    plsc.subcore_barrier()

    @pl.when(sid == 0)
    def _():
        pltpu.sync_copy(acc, s_out.at[cid])


@functools.partial(
    pl.kernel,
    out_type=(
        jax.ShapeDtypeStruct((E,), jnp.float32),
        jax.ShapeDtypeStruct((NC, N), jnp.float32),
    ),
    mesh=_MESH,
    compiler_params=_SC_PARAMS,
    scratch_types=[
        pltpu.VMEM((EPT,), jnp.int32),
        pltpu.VMEM((EPT,), jnp.int32),
        pltpu.VMEM((N,), jnp.float32),
        pltpu.VMEM((N,), jnp.float32),
        pltpu.VMEM((EPT,), jnp.float32),
        pltpu.VMEM_SHARED((N,), jnp.float32),
    ],
)
def _sc_attn(row_hbm, col_hbm, a_hbm, b_hbm, zeros_hbm, mw_out, deg_out,
             roww, colw, av, bv, mwv, acc):
    cid = lax.axis_index("c")
    sid = lax.axis_index("s")
    base = _wid(cid, sid) * EPT
    pltpu.sync_copy(row_hbm.at[pl.ds(base, EPT)], roww)
    pltpu.sync_copy(col_hbm.at[pl.ds(base, EPT)], colw)
    pltpu.sync_copy(a_hbm, av)
    pltpu.sync_copy(b_hbm, bv)

    @pl.when(sid == 0)
    def _():
        pltpu.sync_copy(zeros_hbm, acc)

    def step(i, carry):
        r = roww[pl.ds(i * 16, 16)]
        c = colw[pl.ds(i * 16, 16)]
        aa = plsc.load_gather(av, [r])
        bb = plsc.load_gather(bv, [c])
        w = jnp.maximum(aa + bb, 0.0)
        sig = 1.0 / (1.0 + jnp.exp(-w))
        m = jnp.minimum(sig * ZETA, 1.0)
        mwv[pl.ds(i * 16, 16)] = m * w
        return carry

    lax.fori_loop(0, EPT // 16, step, 0)
    pltpu.sync_copy(mwv, mw_out.at[pl.ds(base, EPT)])
    plsc.subcore_barrier()
    pltpu.sync_copy(mwv, acc.at[colw], add=True)
    plsc.subcore_barrier()

    @pl.when(sid == 0)
    def _():
        pltpu.sync_copy(acc, deg_out.at[cid])


@functools.partial(
    pl.kernel,
    out_type=jax.ShapeDtypeStruct((NC, N, H1), jnp.float32),
    mesh=_MESH,
    compiler_params=_SC_PARAMS,
    scratch_types=[
        pltpu.VMEM((WIN,), jnp.int32),
        pltpu.VMEM((WIN,), jnp.int32),
        pltpu.VMEM((WIN,), jnp.float32),
        pltpu.VMEM((WIN, H1), jnp.float32),
        pltpu.VMEM((WIN,), jnp.int32),
        pltpu.VMEM((WIN,), jnp.int32),
        pltpu.VMEM((WIN,), jnp.float32),
        pltpu.VMEM((WIN, H1), jnp.float32),
        pltpu.VMEM_SHARED((N, H1), jnp.float32),
        pltpu.SemaphoreType.DMA,
        pltpu.SemaphoreType.DMA,
    ],
)
def _sc_agg8(row_hbm, col_hbm, mw_hbm, tmp2_hbm, zeros_hbm, s2_out, roww0,
             colw0, mww0, rows0, roww1, colw1, mww1, rows1, acc, sem0, sem1):
    cid = lax.axis_index("c")
    sid = lax.axis_index("s")
    base = _wid(cid, sid) * EPT
    rw = [roww0, roww1]
    cw = [colw0, colw1]
    mwb = [mww0, mww1]
    rb = [rows0, rows1]
    sems = [sem0, sem1]

    @pl.when(sid == 0)
    def _():
        pltpu.sync_copy(zeros_hbm, acc)

    plsc.subcore_barrier()
    iota16 = lax.iota(jnp.int32, 16)
    hi = iota16 >> 3
    cidx = iota16 & 7
    pltpu.sync_copy(row_hbm.at[pl.ds(base, WIN)], roww0)
    copies = {0: pltpu.async_copy(tmp2_hbm.at[roww0], rows0, sem0)}
    pltpu.sync_copy(col_hbm.at[pl.ds(base, WIN)], colw0)
    pltpu.sync_copy(mw_hbm.at[pl.ds(base, WIN)], mww0)
    for w in range(NWIN):
        cur, nxt = w % 2, (w + 1) % 2
        if w + 1 < NWIN:
            b1 = base + (w + 1) * WIN
            pltpu.sync_copy(row_hbm.at[pl.ds(b1, WIN)], rw[nxt])
            copies[w + 1] = pltpu.async_copy(tmp2_hbm.at[rw[nxt]], rb[nxt],
                                             sems[nxt])
            pltpu.sync_copy(col_hbm.at[pl.ds(b1, WIN)], cw[nxt])
            pltpu.sync_copy(mw_hbm.at[pl.ds(b1, WIN)], mwb[nxt])
        copies[w].wait()
        rows, mww = rb[cur], mwb[cur]

        def step(j, carry):
            rbase = hi + 8 * j
            for u in range(4):
                rid = rbase + 2 * u
                v = plsc.load_gather(rows, [rid, cidx])
                s = plsc.load_gather(mww, [rid])
                plsc.store_scatter(rows, [rid, cidx], v * s)
            return carry

        lax.fori_loop(0, WIN // 8, step, 0)
        pltpu.sync_copy(rows, acc.at[cw[cur]], add=True)
    plsc.subcore_barrier()

    @pl.when(sid == 0)
    def _():
        pltpu.sync_copy(acc, s2_out.at[cid])


def _tc1_body(x_ref, w0_ref, degp_ref, tmp_ref, dinv_ref):
    deg = degp_ref[0] + degp_ref[1] + 1.0
    dinv = lax.rsqrt(deg)
    xw = jnp.dot(x_ref[...], w0_ref[...], preferred_element_type=jnp.float32)
    tmp_ref[...] = xw * dinv[:, None]
    dinv_ref[...] = dinv


def _tc1(x, w0, degp):
    return pl.pallas_call(
        _tc1_body,
        out_shape=(
            jax.ShapeDtypeStruct((N, H0), jnp.float32),
            jax.ShapeDtypeStruct((N,), jnp.float32),
        ),
    )(x, w0, degp)


def _tc2_body(sp_ref, tmp_ref, dinv_ref, b0_ref, wnb_ref, wself_ref,
              watt_ref, batt_ref, bnb_ref, bself_ref, w1_ref,
              a_ref, b_ref, hw1_ref):
    h = dinv_ref[...][:, None] * (sp_ref[0] + sp_ref[1] + tmp_ref[...])
    h = h + b0_ref[...][None, :]
    watt = watt_ref[...]
    u = jnp.dot(wnb_ref[...], watt[:H1], preferred_element_type=jnp.float32)
    v = jnp.dot(wself_ref[...], watt[H1:], preferred_element_type=jnp.float32)
    proj = jnp.concatenate([u, v, w1_ref[...]], axis=1)
    y = jnp.dot(h, proj, preferred_element_type=jnp.float32)
    c0 = (jnp.sum(bnb_ref[...] * watt[:H1, 0])
          + jnp.sum(bself_ref[...] * watt[H1:, 0]) + batt_ref[0])
    a_ref[...] = y[:, 0] + c0
    b_ref[...] = y[:, 1]
    hw1_ref[...] = y[:, 2:]


def _tc2(sp, tmp, dinv1, b0, wnb, wself, watt, batt, bnb, bself, w1):
    return pl.pallas_call(
        _tc2_body,
        out_shape=(
            jax.ShapeDtypeStruct((N,), jnp.float32),
            jax.ShapeDtypeStruct((N,), jnp.float32),
            jax.ShapeDtypeStruct((N, H1), jnp.float32),
        ),
    )(sp, tmp, dinv1, b0, wnb, wself, watt, batt, bnb, bself, w1)


def _tc3_body(degp_ref, hw1_ref, tmp2_ref, dinv_ref):
    deg = degp_ref[0] + degp_ref[1] + 1.0
    dinv = lax.rsqrt(deg)
    tmp2_ref[...] = hw1_ref[...] * dinv[:, None]
    dinv_ref[...] = dinv


def _tc3(deg2p, hw1):
    return pl.pallas_call(
        _tc3_body,
        out_shape=(
            jax.ShapeDtypeStruct((N, H1), jnp.float32),
            jax.ShapeDtypeStruct((N,), jnp.float32),
        ),
    )(deg2p, hw1)


def _tc4_body(s2p_ref, tmp2_ref, dinv_ref, b1_ref, out_ref):
    agg = s2p_ref[0] + s2p_ref[1] + tmp2_ref[...]
    out_ref[...] = dinv_ref[...][:, None] * agg + b1_ref[...][None, :]


def _tc4(s2p, tmp2, dinv2, b1):
    return pl.pallas_call(
        _tc4_body,
        out_shape=jax.ShapeDtypeStruct((N, H1), jnp.float32),
    )(s2p, tmp2, dinv2, b1)


def kernel(x, edge_index, W0, b0, W1, b1, Wnb, bnb, Wself, bself, Watt, batt):
    row = edge_index[0].astype(jnp.int32)
    col = edge_index[1].astype(jnp.int32)
    ones_n = jnp.ones((N,), jnp.float32)
    zeros_n = jnp.zeros((N,), jnp.float32)
    zeros_n32 = jnp.zeros((N, H0), jnp.float32)
    zeros_n8 = jnp.zeros((N, H1), jnp.float32)

    degp = _sc_deg(col, ones_n, zeros_n)
    tmp, dinv1 = _tc1(x, W0, degp)
    sp = _sc_agg32(row, col, tmp, zeros_n32)
    av, bv, hw1 = _tc2(sp, tmp, dinv1, b0, Wnb, Wself, Watt, batt, bnb,
                       bself, W1)
    mw, deg2p = _sc_attn(row, col, av, bv, zeros_n)
    tmp2, dinv2 = _tc3(deg2p, hw1)
    s2p = _sc_agg8(row, col, mw, tmp2, zeros_n8)
    return _tc4(s2p, tmp2, dinv2, b1)


# speedup vs baseline: 1.0560x; 1.0560x over previous
import functools

import jax
import jax.numpy as jnp
from jax import lax
from jax.experimental import pallas as pl
from jax.experimental.pallas import tpu as pltpu
from jax.experimental.pallas import tpu_sc as plsc

N = 10000
E = 320000
D_IN = 128
H0 = 32
H1 = 8
ZETA = 1.01

NC = 2
NS = 16
NW = NC * NS
EPT = E // NW
WIN_B = 1000
NWIN_B = EPT // WIN_B
WIN = 2000
NWIN = EPT // WIN

_MESH = plsc.VectorSubcoreMesh(
    core_axis_name="c", subcore_axis_name="s", num_cores=NC, num_subcores=NS
)
_SC_PARAMS = pltpu.CompilerParams(
    use_tc_tiling_on_sc=False, needs_layout_passes=False
)


def _wid(cid, sid):
    return cid * NS + sid


@functools.partial(
    pl.kernel,
    out_type=jax.ShapeDtypeStruct((NC, N), jnp.float32),
    mesh=_MESH,
    compiler_params=_SC_PARAMS,
    scratch_types=[
        pltpu.VMEM((EPT,), jnp.int32),
        pltpu.VMEM((N,), jnp.float32),
        pltpu.VMEM_SHARED((N,), jnp.float32),
    ],
)
def _sc_deg(col_hbm, ones_hbm, zeros_hbm, deg_out, colw, onesv, acc):
    cid = lax.axis_index("c")
    sid = lax.axis_index("s")
    base = _wid(cid, sid) * EPT
    pltpu.sync_copy(col_hbm.at[pl.ds(base, EPT)], colw)
    pltpu.sync_copy(ones_hbm, onesv)

    @pl.when(sid == 0)
    def _():
        pltpu.sync_copy(zeros_hbm, acc)

    plsc.subcore_barrier()
    pltpu.sync_copy(onesv, acc.at[colw], add=True)
    plsc.subcore_barrier()

    @pl.when(sid == 0)
    def _():
        pltpu.sync_copy(acc, deg_out.at[cid])


@functools.partial(
    pl.kernel,
    out_type=jax.ShapeDtypeStruct((NC, N, H0), jnp.float32),
    mesh=_MESH,
    compiler_params=_SC_PARAMS,
    scratch_types=[
        pltpu.VMEM((WIN_B,), jnp.int32),
        pltpu.VMEM((WIN_B,), jnp.int32),
        pltpu.VMEM((WIN_B, H0), jnp.float32),
        pltpu.VMEM((WIN_B,), jnp.int32),
        pltpu.VMEM((WIN_B,), jnp.int32),
        pltpu.VMEM((WIN_B, H0), jnp.float32),
        pltpu.VMEM_SHARED((N, H0), jnp.float32),
        pltpu.SemaphoreType.DMA,
        pltpu.SemaphoreType.DMA,
    ],
)
def _sc_agg32(row_hbm, col_hbm, tmp_hbm, zeros_hbm, s_out, roww0, colw0,
              rows0, roww1, colw1, rows1, acc, sem0, sem1):
    cid = lax.axis_index("c")
    sid = lax.axis_index("s")
    base = _wid(cid, sid) * EPT
    rw = [roww0, roww1]
    cw = [colw0, colw1]
    rb = [rows0, rows1]
    sems = [sem0, sem1]

    @pl.when(sid == 0)
    def _():
        pltpu.sync_copy(zeros_hbm, acc)

    plsc.subcore_barrier()
    pltpu.sync_copy(row_hbm.at[pl.ds(base, WIN_B)], roww0)
    copies = {0: pltpu.async_copy(tmp_hbm.at[roww0], rows0, sem0)}
    pltpu.sync_copy(col_hbm.at[pl.ds(base, WIN_B)], colw0)
    for w in range(NWIN_B):
        cur, nxt = w % 2, (w + 1) % 2
        if w + 1 < NWIN_B:
            b1 = base + (w + 1) * WIN_B
            pltpu.sync_copy(row_hbm.at[pl.ds(b1, WIN_B)], rw[nxt])
            copies[w + 1] = pltpu.async_copy(tmp_hbm.at[rw[nxt]], rb[nxt],
                                             sems[nxt])
            pltpu.sync_copy(col_hbm.at[pl.ds(b1, WIN_B)], cw[nxt])
        copies[w].wait()
        pltpu.sync_copy(rb[cur], acc.at[cw[cur]], add=True)
    plsc.subcore_barrier()

    @pl.when(sid == 0)
    def _():
        pltpu.sync_copy(acc, s_out.at[cid])


@functools.partial(
    pl.kernel,
    out_type=(
        jax.ShapeDtypeStruct((E,), jnp.float32),
        jax.ShapeDtypeStruct((NC, N), jnp.float32),
    ),
    mesh=_MESH,
    compiler_params=_SC_PARAMS,
    scratch_types=[
        pltpu.VMEM((EPT,), jnp.int32),
        pltpu.VMEM((EPT,), jnp.int32),
        pltpu.VMEM((N,), jnp.float32),
        pltpu.VMEM((N,), jnp.float32),
        pltpu.VMEM((EPT,), jnp.float32),
        pltpu.VMEM_SHARED((N,), jnp.float32),
    ],
)
def _sc_attn(row_hbm, col_hbm, a_hbm, b_hbm, zeros_hbm, mw_out, deg_out,
             roww, colw, av, bv, mwv, acc):
    cid = lax.axis_index("c")
    sid = lax.axis_index("s")
    base = _wid(cid, sid) * EPT
    pltpu.sync_copy(row_hbm.at[pl.ds(base, EPT)], roww)
    pltpu.sync_copy(col_hbm.at[pl.ds(base, EPT)], colw)
    pltpu.sync_copy(a_hbm, av)
    pltpu.sync_copy(b_hbm, bv)

    @pl.when(sid == 0)
    def _():
        pltpu.sync_copy(zeros_hbm, acc)

    def step(i, carry):
        r = roww[pl.ds(i * 16, 16)]
        c = colw[pl.ds(i * 16, 16)]
        aa = plsc.load_gather(av, [r])
        bb = plsc.load_gather(bv, [c])
        w = jnp.maximum(aa + bb, 0.0)
        sig = 1.0 / (1.0 + jnp.exp(-w))
        m = jnp.minimum(sig * ZETA, 1.0)
        mwv[pl.ds(i * 16, 16)] = m * w
        return carry

    lax.fori_loop(0, EPT // 16, step, 0)
    pltpu.sync_copy(mwv, mw_out.at[pl.ds(base, EPT)])
    plsc.subcore_barrier()
    pltpu.sync_copy(mwv, acc.at[colw], add=True)
    plsc.subcore_barrier()

    @pl.when(sid == 0)
    def _():
        pltpu.sync_copy(acc, deg_out.at[cid])


@functools.partial(
    pl.kernel,
    out_type=jax.ShapeDtypeStruct((NC, N, H1), jnp.float32),
    mesh=_MESH,
    compiler_params=_SC_PARAMS,
    scratch_types=[
        pltpu.VMEM((WIN,), jnp.int32),
        pltpu.VMEM((WIN,), jnp.int32),
        pltpu.VMEM((WIN,), jnp.float32),
        pltpu.VMEM((WIN, H1), jnp.float32),
        pltpu.VMEM((WIN,), jnp.int32),
        pltpu.VMEM((WIN,), jnp.int32),
        pltpu.VMEM((WIN,), jnp.float32),
        pltpu.VMEM((WIN, H1), jnp.float32),
        pltpu.VMEM_SHARED((N, H1), jnp.float32),
        pltpu.SemaphoreType.DMA,
        pltpu.SemaphoreType.DMA,
    ],
)
def _sc_agg8(row_hbm, col_hbm, mw_hbm, tmp2_hbm, zeros_hbm, s2_out, roww0,
             colw0, mww0, rows0, roww1, colw1, mww1, rows1, acc, sem0, sem1):
    cid = lax.axis_index("c")
    sid = lax.axis_index("s")
    base = _wid(cid, sid) * EPT
    rw = [roww0, roww1]
    cw = [colw0, colw1]
    mwb = [mww0, mww1]
    rb = [rows0, rows1]
    sems = [sem0, sem1]

    @pl.when(sid == 0)
    def _():
        pltpu.sync_copy(zeros_hbm, acc)

    plsc.subcore_barrier()
    iota16 = lax.iota(jnp.int32, 16)
    hi = iota16 >> 3
    cidx = iota16 & 7
    pltpu.sync_copy(row_hbm.at[pl.ds(base, WIN)], roww0)
    copies = {0: pltpu.async_copy(tmp2_hbm.at[roww0], rows0, sem0)}
    pltpu.sync_copy(col_hbm.at[pl.ds(base, WIN)], colw0)
    pltpu.sync_copy(mw_hbm.at[pl.ds(base, WIN)], mww0)
    for w in range(NWIN):
        cur, nxt = w % 2, (w + 1) % 2
        if w + 1 < NWIN:
            b1 = base + (w + 1) * WIN
            pltpu.sync_copy(row_hbm.at[pl.ds(b1, WIN)], rw[nxt])
            copies[w + 1] = pltpu.async_copy(tmp2_hbm.at[rw[nxt]], rb[nxt],
                                             sems[nxt])
            pltpu.sync_copy(col_hbm.at[pl.ds(b1, WIN)], cw[nxt])
            pltpu.sync_copy(mw_hbm.at[pl.ds(b1, WIN)], mwb[nxt])
        copies[w].wait()
        rows, mww = rb[cur], mwb[cur]

        def step(j, carry):
            rid0 = hi + 4 * j
            rid1 = rid0 + 2
            v0 = plsc.load_gather(rows, [rid0, cidx])
            s0 = plsc.load_gather(mww, [rid0])
            v1 = plsc.load_gather(rows, [rid1, cidx])
            s1 = plsc.load_gather(mww, [rid1])
            plsc.store_scatter(rows, [rid0, cidx], v0 * s0)
            plsc.store_scatter(rows, [rid1, cidx], v1 * s1)
            return carry

        lax.fori_loop(0, WIN // 4, step, 0)
        pltpu.sync_copy(rows, acc.at[cw[cur]], add=True)
    plsc.subcore_barrier()

    @pl.when(sid == 0)
    def _():
        pltpu.sync_copy(acc, s2_out.at[cid])


def _tc1_body(x_ref, w0_ref, degp_ref, tmp_ref, dinv_ref):
    deg = degp_ref[0] + degp_ref[1] + 1.0
    dinv = lax.rsqrt(deg)
    xw = jnp.dot(x_ref[...], w0_ref[...], preferred_element_type=jnp.float32)
    tmp_ref[...] = xw * dinv[:, None]
    dinv_ref[...] = dinv


def _tc1(x, w0, degp):
    return pl.pallas_call(
        _tc1_body,
        out_shape=(
            jax.ShapeDtypeStruct((N, H0), jnp.float32),
            jax.ShapeDtypeStruct((N,), jnp.float32),
        ),
    )(x, w0, degp)


def _tc2_body(sp_ref, tmp_ref, dinv_ref, b0_ref, wnb_ref, wself_ref,
              watt_ref, batt_ref, bnb_ref, bself_ref, w1_ref,
              a_ref, b_ref, hw1_ref):
    h = dinv_ref[...][:, None] * (sp_ref[0] + sp_ref[1] + tmp_ref[...])
    h = h + b0_ref[...][None, :]
    watt = watt_ref[...]
    u = jnp.dot(wnb_ref[...], watt[:H1], preferred_element_type=jnp.float32)
    v = jnp.dot(wself_ref[...], watt[H1:], preferred_element_type=jnp.float32)
    proj = jnp.concatenate([u, v, w1_ref[...]], axis=1)
    y = jnp.dot(h, proj, preferred_element_type=jnp.float32)
    c0 = (jnp.sum(bnb_ref[...] * watt[:H1, 0])
          + jnp.sum(bself_ref[...] * watt[H1:, 0]) + batt_ref[0])
    a_ref[...] = y[:, 0] + c0
    b_ref[...] = y[:, 1]
    hw1_ref[...] = y[:, 2:]


def _tc2(sp, tmp, dinv1, b0, wnb, wself, watt, batt, bnb, bself, w1):
    return pl.pallas_call(
        _tc2_body,
        out_shape=(
            jax.ShapeDtypeStruct((N,), jnp.float32),
            jax.ShapeDtypeStruct((N,), jnp.float32),
            jax.ShapeDtypeStruct((N, H1), jnp.float32),
        ),
    )(sp, tmp, dinv1, b0, wnb, wself, watt, batt, bnb, bself, w1)


def _tc3_body(degp_ref, hw1_ref, tmp2_ref, dinv_ref):
    deg = degp_ref[0] + degp_ref[1] + 1.0
    dinv = lax.rsqrt(deg)
    tmp2_ref[...] = hw1_ref[...] * dinv[:, None]
    dinv_ref[...] = dinv


def _tc3(deg2p, hw1):
    return pl.pallas_call(
        _tc3_body,
        out_shape=(
            jax.ShapeDtypeStruct((N, H1), jnp.float32),
            jax.ShapeDtypeStruct((N,), jnp.float32),
        ),
    )(deg2p, hw1)


def _tc4_body(s2p_ref, tmp2_ref, dinv_ref, b1_ref, out_ref):
    agg = s2p_ref[0] + s2p_ref[1] + tmp2_ref[...]
    out_ref[...] = dinv_ref[...][:, None] * agg + b1_ref[...][None, :]


def _tc4(s2p, tmp2, dinv2, b1):
    return pl.pallas_call(
        _tc4_body,
        out_shape=jax.ShapeDtypeStruct((N, H1), jnp.float32),
    )(s2p, tmp2, dinv2, b1)


def kernel(x, edge_index, W0, b0, W1, b1, Wnb, bnb, Wself, bself, Watt, batt):
    row = edge_index[0].astype(jnp.int32)
    col = edge_index[1].astype(jnp.int32)
    ones_n = jnp.ones((N,), jnp.float32)
    zeros_n = jnp.zeros((N,), jnp.float32)
    zeros_n32 = jnp.zeros((N, H0), jnp.float32)
    zeros_n8 = jnp.zeros((N, H1), jnp.float32)

    degp = _sc_deg(col, ones_n, zeros_n)
    tmp, dinv1 = _tc1(x, W0, degp)
    sp = _sc_agg32(row, col, tmp, zeros_n32)
    av, bv, hw1 = _tc2(sp, tmp, dinv1, b0, Wnb, Wself, Watt, batt, bnb,
                       bself, W1)
    mw, deg2p = _sc_attn(row, col, av, bv, zeros_n)
    tmp2, dinv2 = _tc3(deg2p, hw1)
    s2p = _sc_agg8(row, col, mw, tmp2, zeros_n8)
    return _tc4(s2p, tmp2, dinv2, b1)


# speedup vs baseline: 1.0876x; 1.0298x over previous
import functools

import jax
import jax.numpy as jnp
from jax import lax
from jax.experimental import pallas as pl
from jax.experimental.pallas import tpu as pltpu
from jax.experimental.pallas import tpu_sc as plsc

N = 10000
E = 320000
D_IN = 128
H0 = 32
H1 = 8
ZETA = 1.01

NC = 2
NS = 16
NW = NC * NS
EPT = E // NW
WIN_B = 1000
NWIN_B = EPT // WIN_B
WIN = 2000
NWIN = EPT // WIN

_MESH = plsc.VectorSubcoreMesh(
    core_axis_name="c", subcore_axis_name="s", num_cores=NC, num_subcores=NS
)
_SC_PARAMS = pltpu.CompilerParams(
    use_tc_tiling_on_sc=False, needs_layout_passes=False
)


def _wid(cid, sid):
    return cid * NS + sid


@functools.partial(
    pl.kernel,
    out_type=jax.ShapeDtypeStruct((NC, N), jnp.float32),
    mesh=_MESH,
    compiler_params=_SC_PARAMS,
    scratch_types=[
        pltpu.VMEM((EPT,), jnp.int32),
        pltpu.VMEM((N,), jnp.float32),
        pltpu.VMEM_SHARED((N,), jnp.float32),
    ],
)
def _sc_deg(col_hbm, zeros_hbm, deg_out, colw, onesv, acc):
    cid = lax.axis_index("c")
    sid = lax.axis_index("s")
    base = _wid(cid, sid) * EPT
    pltpu.sync_copy(col_hbm.at[pl.ds(base, EPT)], colw)
    ones16 = jnp.full((16,), 1.0, dtype=jnp.float32)

    def fill(i, carry):
        onesv[pl.ds(i * 16, 16)] = ones16
        return carry

    lax.fori_loop(0, EPT // 16, fill, 0)

    @pl.when(sid == 0)
    def _():
        pltpu.sync_copy(zeros_hbm, acc)

    plsc.subcore_barrier()
    pltpu.sync_copy(onesv, acc.at[colw], add=True)
    plsc.subcore_barrier()

    @pl.when(sid == 0)
    def _():
        pltpu.sync_copy(acc, deg_out.at[cid])


@functools.partial(
    pl.kernel,
    out_type=jax.ShapeDtypeStruct((NC, N, H0), jnp.float32),
    mesh=_MESH,
    compiler_params=_SC_PARAMS,
    scratch_types=[
        pltpu.VMEM((WIN_B,), jnp.int32),
        pltpu.VMEM((WIN_B,), jnp.int32),
        pltpu.VMEM((WIN_B, H0), jnp.float32),
        pltpu.VMEM((WIN_B,), jnp.int32),
        pltpu.VMEM((WIN_B,), jnp.int32),
        pltpu.VMEM((WIN_B, H0), jnp.float32),
        pltpu.VMEM_SHARED((N, H0), jnp.float32),
        pltpu.SemaphoreType.DMA,
        pltpu.SemaphoreType.DMA,
    ],
)
def _sc_agg32(row_hbm, col_hbm, tmp_hbm, zeros_hbm, s_out, roww0, colw0,
              rows0, roww1, colw1, rows1, acc, sem0, sem1):
    cid = lax.axis_index("c")
    sid = lax.axis_index("s")
    base = _wid(cid, sid) * EPT
    rw = [roww0, roww1]
    cw = [colw0, colw1]
    rb = [rows0, rows1]
    sems = [sem0, sem1]

    @pl.when(sid == 0)
    def _():
        pltpu.sync_copy(zeros_hbm, acc)

    plsc.subcore_barrier()
    pltpu.sync_copy(row_hbm.at[pl.ds(base, WIN_B)], roww0)
    copies = {0: pltpu.async_copy(tmp_hbm.at[roww0], rows0, sem0)}
    pltpu.sync_copy(col_hbm.at[pl.ds(base, WIN_B)], colw0)
    for w in range(NWIN_B):
        cur, nxt = w % 2, (w + 1) % 2
        if w + 1 < NWIN_B:
            b1 = base + (w + 1) * WIN_B
            pltpu.sync_copy(row_hbm.at[pl.ds(b1, WIN_B)], rw[nxt])
            copies[w + 1] = pltpu.async_copy(tmp_hbm.at[rw[nxt]], rb[nxt],
                                             sems[nxt])
            pltpu.sync_copy(col_hbm.at[pl.ds(b1, WIN_B)], cw[nxt])
        copies[w].wait()
        pltpu.sync_copy(rb[cur], acc.at[cw[cur]], add=True)
    plsc.subcore_barrier()

    @pl.when(sid == 0)
    def _():
        pltpu.sync_copy(acc, s_out.at[cid])


@functools.partial(
    pl.kernel,
    out_type=(
        jax.ShapeDtypeStruct((E,), jnp.float32),
        jax.ShapeDtypeStruct((NC, N), jnp.float32),
    ),
    mesh=_MESH,
    compiler_params=_SC_PARAMS,
    scratch_types=[
        pltpu.VMEM((EPT,), jnp.int32),
        pltpu.VMEM((EPT,), jnp.int32),
        pltpu.VMEM((N,), jnp.float32),
        pltpu.VMEM((N,), jnp.float32),
        pltpu.VMEM((EPT,), jnp.float32),
        pltpu.VMEM_SHARED((N,), jnp.float32),
        pltpu.VMEM_SHARED((N,), jnp.float32),
        pltpu.VMEM_SHARED((N,), jnp.float32),
        pltpu.SemaphoreType.DMA,
    ],
)
def _sc_attn(row_hbm, col_hbm, a_hbm, b_hbm, zeros_hbm, mw_out, deg_out,
             roww, colw, av, bv, mwv, acc, sa, sb, semo):
    cid = lax.axis_index("c")
    sid = lax.axis_index("s")
    base = _wid(cid, sid) * EPT

    @pl.when(sid == 0)
    def _():
        pltpu.sync_copy(a_hbm, sa)

    @pl.when(sid == 1)
    def _():
        pltpu.sync_copy(b_hbm, sb)

    @pl.when(sid == 2)
    def _():
        pltpu.sync_copy(zeros_hbm, acc)

    pltpu.sync_copy(row_hbm.at[pl.ds(base, EPT)], roww)
    pltpu.sync_copy(col_hbm.at[pl.ds(base, EPT)], colw)
    plsc.subcore_barrier()
    pltpu.sync_copy(sa, av)
    pltpu.sync_copy(sb, bv)

    def step(i, carry):
        r = roww[pl.ds(i * 16, 16)]
        c = colw[pl.ds(i * 16, 16)]
        aa = plsc.load_gather(av, [r])
        bb = plsc.load_gather(bv, [c])
        w = jnp.maximum(aa + bb, 0.0)
        sig = 1.0 / (1.0 + jnp.exp(-w))
        m = jnp.minimum(sig * ZETA, 1.0)
        mwv[pl.ds(i * 16, 16)] = m * w
        return carry

    lax.fori_loop(0, EPT // 16, step, 0)
    cp_out = pltpu.async_copy(mwv, mw_out.at[pl.ds(base, EPT)], semo)
    pltpu.sync_copy(mwv, acc.at[colw], add=True)
    cp_out.wait()
    plsc.subcore_barrier()

    @pl.when(sid == 0)
    def _():
        pltpu.sync_copy(acc, deg_out.at[cid])


@functools.partial(
    pl.kernel,
    out_type=jax.ShapeDtypeStruct((NC, N, H1), jnp.float32),
    mesh=_MESH,
    compiler_params=_SC_PARAMS,
    scratch_types=[
        pltpu.VMEM((WIN,), jnp.int32),
        pltpu.VMEM((WIN,), jnp.int32),
        pltpu.VMEM((WIN,), jnp.float32),
        pltpu.VMEM((WIN, H1), jnp.float32),
        pltpu.VMEM((WIN,), jnp.int32),
        pltpu.VMEM((WIN,), jnp.int32),
        pltpu.VMEM((WIN,), jnp.float32),
        pltpu.VMEM((WIN, H1), jnp.float32),
        pltpu.VMEM_SHARED((N, H1), jnp.float32),
        pltpu.VMEM_SHARED((N, H1), jnp.float32),
        pltpu.SemaphoreType.DMA,
        pltpu.SemaphoreType.DMA,
    ],
)
def _sc_agg8(row_hbm, col_hbm, mw_hbm, tmp2_hbm, zeros_hbm, s2_out, roww0,
             colw0, mww0, rows0, roww1, colw1, mww1, rows1, acc, stage, sem0,
             sem1):
    cid = lax.axis_index("c")
    sid = lax.axis_index("s")
    base = _wid(cid, sid) * EPT
    rw = [roww0, roww1]
    cw = [colw0, colw1]
    mwb = [mww0, mww1]
    rb = [rows0, rows1]
    sems = [sem0, sem1]

    @pl.when(sid == 0)
    def _():
        pltpu.sync_copy(zeros_hbm, acc)

    @pl.when(sid == 1)
    def _():
        pltpu.sync_copy(tmp2_hbm, stage)

    plsc.subcore_barrier()
    iota16 = lax.iota(jnp.int32, 16)
    hi = iota16 >> 3
    cidx = iota16 & 7
    pltpu.sync_copy(row_hbm.at[pl.ds(base, WIN)], roww0)
    copies = {0: pltpu.async_copy(stage.at[roww0], rows0, sem0)}
    pltpu.sync_copy(col_hbm.at[pl.ds(base, WIN)], colw0)
    pltpu.sync_copy(mw_hbm.at[pl.ds(base, WIN)], mww0)
    for w in range(NWIN):
        cur, nxt = w % 2, (w + 1) % 2
        if w + 1 < NWIN:
            b1 = base + (w + 1) * WIN
            pltpu.sync_copy(row_hbm.at[pl.ds(b1, WIN)], rw[nxt])
            copies[w + 1] = pltpu.async_copy(stage.at[rw[nxt]], rb[nxt],
                                             sems[nxt])
            pltpu.sync_copy(col_hbm.at[pl.ds(b1, WIN)], cw[nxt])
            pltpu.sync_copy(mw_hbm.at[pl.ds(b1, WIN)], mwb[nxt])
        copies[w].wait()
        rows, mww = rb[cur], mwb[cur]

        def step(j, carry):
            rid0 = hi + 4 * j
            rid1 = rid0 + 2
            v0 = plsc.load_gather(rows, [rid0, cidx])
            s0 = plsc.load_gather(mww, [rid0])
            v1 = plsc.load_gather(rows, [rid1, cidx])
            s1 = plsc.load_gather(mww, [rid1])
            plsc.store_scatter(rows, [rid0, cidx], v0 * s0)
            plsc.store_scatter(rows, [rid1, cidx], v1 * s1)
            return carry

        lax.fori_loop(0, WIN // 4, step, 0)
        pltpu.sync_copy(rows, acc.at[cw[cur]], add=True)
    plsc.subcore_barrier()

    @pl.when(sid == 0)
    def _():
        pltpu.sync_copy(acc, s2_out.at[cid])


def _tc1_body(x_ref, w0_ref, degp_ref, tmp_ref, dinv_ref):
    deg = degp_ref[0] + degp_ref[1] + 1.0
    dinv = lax.rsqrt(deg)
    xw = jnp.dot(x_ref[...], w0_ref[...], preferred_element_type=jnp.float32)
    tmp_ref[...] = xw * dinv[:, None]
    dinv_ref[...] = dinv


def _tc1(x, w0, degp):
    return pl.pallas_call(
        _tc1_body,
        out_shape=(
            jax.ShapeDtypeStruct((N, H0), jnp.float32),
            jax.ShapeDtypeStruct((N,), jnp.float32),
        ),
    )(x, w0, degp)


def _tc2_body(sp_ref, tmp_ref, dinv_ref, b0_ref, wnb_ref, wself_ref,
              watt_ref, batt_ref, bnb_ref, bself_ref, w1_ref,
              a_ref, b_ref, hw1_ref):
    h = dinv_ref[...][:, None] * (sp_ref[0] + sp_ref[1] + tmp_ref[...])
    h = h + b0_ref[...][None, :]
    watt = watt_ref[...]
    u = jnp.dot(wnb_ref[...], watt[:H1], preferred_element_type=jnp.float32)
    v = jnp.dot(wself_ref[...], watt[H1:], preferred_element_type=jnp.float32)
    proj = jnp.concatenate([u, v, w1_ref[...]], axis=1)
    y = jnp.dot(h, proj, preferred_element_type=jnp.float32)
    c0 = (jnp.sum(bnb_ref[...] * watt[:H1, 0])
          + jnp.sum(bself_ref[...] * watt[H1:, 0]) + batt_ref[0])
    a_ref[...] = y[:, 0] + c0
    b_ref[...] = y[:, 1]
    hw1_ref[...] = y[:, 2:]


def _tc2(sp, tmp, dinv1, b0, wnb, wself, watt, batt, bnb, bself, w1):
    return pl.pallas_call(
        _tc2_body,
        out_shape=(
            jax.ShapeDtypeStruct((N,), jnp.float32),
            jax.ShapeDtypeStruct((N,), jnp.float32),
            jax.ShapeDtypeStruct((N, H1), jnp.float32),
        ),
    )(sp, tmp, dinv1, b0, wnb, wself, watt, batt, bnb, bself, w1)


def _tc3_body(degp_ref, hw1_ref, tmp2_ref, dinv_ref):
    deg = degp_ref[0] + degp_ref[1] + 1.0
    dinv = lax.rsqrt(deg)
    tmp2_ref[...] = hw1_ref[...] * dinv[:, None]
    dinv_ref[...] = dinv


def _tc3(deg2p, hw1):
    return pl.pallas_call(
        _tc3_body,
        out_shape=(
            jax.ShapeDtypeStruct((N, H1), jnp.float32),
            jax.ShapeDtypeStruct((N,), jnp.float32),
        ),
    )(deg2p, hw1)


def _tc4_body(s2p_ref, tmp2_ref, dinv_ref, b1_ref, out_ref):
    agg = s2p_ref[0] + s2p_ref[1] + tmp2_ref[...]
    out_ref[...] = dinv_ref[...][:, None] * agg + b1_ref[...][None, :]


def _tc4(s2p, tmp2, dinv2, b1):
    return pl.pallas_call(
        _tc4_body,
        out_shape=jax.ShapeDtypeStruct((N, H1), jnp.float32),
    )(s2p, tmp2, dinv2, b1)


def kernel(x, edge_index, W0, b0, W1, b1, Wnb, bnb, Wself, bself, Watt, batt):
    row = edge_index[0].astype(jnp.int32)
    col = edge_index[1].astype(jnp.int32)
    zeros_n = jnp.zeros((N,), jnp.float32)
    zeros_n32 = jnp.zeros((N, H0), jnp.float32)
    zeros_n8 = jnp.zeros((N, H1), jnp.float32)

    degp = _sc_deg(col, zeros_n)
    tmp, dinv1 = _tc1(x, W0, degp)
    sp = _sc_agg32(row, col, tmp, zeros_n32)
    av, bv, hw1 = _tc2(sp, tmp, dinv1, b0, Wnb, Wself, Watt, batt, bnb,
                       bself, W1)
    mw, deg2p = _sc_attn(row, col, av, bv, zeros_n)
    tmp2, dinv2 = _tc3(deg2p, hw1)
    s2p = _sc_agg8(row, col, mw, tmp2, zeros_n8)
    return _tc4(s2p, tmp2, dinv2, b1)
